# Initial kernel scaffold; baseline (speedup 1.0000x reference)
import jax
import jax.numpy as jnp
from jax.experimental import pallas as pl


def kernel(x, forward_shuffle_idx):
    raise NotImplementedError("write your pallas kernel here")


# speedup vs baseline: 2.2524x; 2.2524x over previous
import functools

import jax
import jax.numpy as jnp
from jax import lax
from jax.experimental import pallas as pl
from jax.experimental.pallas import tpu as pltpu
from jax.experimental.pallas import tpu_sc as plsc

B, S, D = 64, 1024, 768
NW = 32
ROWS = B * S
RPW = ROWS // NW
C = 64
NCHUNK = RPW // C
NBUF = 2


def _sc_permute_gather(table, gidx):
    mesh = plsc.VectorSubcoreMesh(core_axis_name="c", subcore_axis_name="s")

    @functools.partial(
        pl.kernel,
        mesh=mesh,
        out_type=jax.ShapeDtypeStruct((ROWS, D), jnp.float32),
        scratch_types=[
            pltpu.VMEM((RPW,), jnp.int32),
            pltpu.VMEM((C, D), jnp.float32),
            pltpu.VMEM((C, D), jnp.float32),
            pltpu.SemaphoreType.DMA,
            pltpu.SemaphoreType.DMA,
        ],
    )
    def k(table_hbm, gidx_hbm, out_hbm, idx_all, rows0, rows1, sem0, sem1):
        rows = (rows0, rows1)
        sems = (sem0, sem1)
        wid = lax.axis_index("s") * 2 + lax.axis_index("c")
        base = wid * RPW
        pltpu.sync_copy(gidx_hbm.at[pl.ds(base, RPW)], idx_all)
        for b in range(NBUF):
            pltpu.async_copy(
                table_hbm.at[idx_all.at[pl.ds(b * C, C)]], rows[b], sems[b]
            )

        def step(g, carry):
            for b in range(NBUF):
                c = g * NBUF + b
                pltpu.make_async_copy(
                    table_hbm.at[idx_all.at[pl.ds(0, C)]], rows[b], sems[b]
                ).wait()
                pltpu.sync_copy(rows[b], out_hbm.at[pl.ds(base + c * C, C)])

                nxt = c + NBUF

                @pl.when(nxt < NCHUNK)
                def _():
                    pltpu.async_copy(
                        table_hbm.at[idx_all.at[pl.ds(nxt * C, C)]],
                        rows[b],
                        sems[b],
                    )
            return carry

        lax.fori_loop(0, NCHUNK // NBUF, step, 0)

    return k(table, gidx)


def kernel(x, forward_shuffle_idx):
    idx = forward_shuffle_idx.astype(jnp.int32)
    table = x.reshape(ROWS, D)
    gidx = (jnp.arange(B, dtype=jnp.int32)[:, None] * S + idx[None, :]).reshape(-1)
    out = _sc_permute_gather(table, gidx)
    return out.reshape(B, S, D)


# speedup vs baseline: 2.2608x; 1.0037x over previous
import functools

import jax
import jax.numpy as jnp
from jax import lax
from jax.experimental import pallas as pl
from jax.experimental.pallas import tpu as pltpu
from jax.experimental.pallas import tpu_sc as plsc

B, S, D = 64, 1024, 768
NW = 32
ROWS = B * S
RPW = ROWS // NW
C = 32
NCHUNK = RPW // C
NBUF = 4
LOOKAHEAD = 2


def _sc_permute_gather(table, gidx):
    mesh = plsc.VectorSubcoreMesh(core_axis_name="c", subcore_axis_name="s")

    @functools.partial(
        pl.kernel,
        mesh=mesh,
        out_type=jax.ShapeDtypeStruct((ROWS, D), jnp.float32),
        scratch_types=[
            pltpu.VMEM((RPW,), jnp.int32),
            [pltpu.VMEM((C, D), jnp.float32) for _ in range(NBUF)],
            [pltpu.SemaphoreType.DMA for _ in range(NBUF)],
            [pltpu.SemaphoreType.DMA for _ in range(NBUF)],
        ],
    )
    def k(table_hbm, gidx_hbm, out_hbm, idx_all, rows, gsem, ssem):
        wid = lax.axis_index("s") * 2 + lax.axis_index("c")
        base = wid * RPW
        pltpu.sync_copy(gidx_hbm.at[pl.ds(base, RPW)], idx_all)

        def gather(c, b):
            pltpu.async_copy(
                table_hbm.at[idx_all.at[pl.ds(c * C, C)]], rows[b], gsem[b]
            )

        def drain_scatter(b):
            pltpu.make_async_copy(
                rows[b], out_hbm.at[pl.ds(base, C)], ssem[b]
            ).wait()

        for b in range(LOOKAHEAD):
            gather(b, b)

        def step(g, carry):
            for b in range(NBUF):
                u = g * NBUF + b
                nb = (b + LOOKAHEAD) % NBUF

                @pl.when(u + LOOKAHEAD < NCHUNK)
                def _():
                    @pl.when(u + LOOKAHEAD >= NBUF)
                    def _():
                        drain_scatter(nb)

                    gather(u + LOOKAHEAD, nb)

                pltpu.make_async_copy(
                    table_hbm.at[idx_all.at[pl.ds(0, C)]], rows[b], gsem[b]
                ).wait()
                pltpu.async_copy(
                    rows[b], out_hbm.at[pl.ds(base + u * C, C)], ssem[b]
                )
            return carry

        lax.fori_loop(0, NCHUNK // NBUF, step, 0)

        for b in range(NBUF):
            drain_scatter(b)

    return k(table, gidx)


def kernel(x, forward_shuffle_idx):
    idx = forward_shuffle_idx.astype(jnp.int32)
    table = x.reshape(ROWS, D)
    gidx = (jnp.arange(B, dtype=jnp.int32)[:, None] * S + idx[None, :]).reshape(-1)
    out = _sc_permute_gather(table, gidx)
    return out.reshape(B, S, D)
